# baseline (speedup 1.0000x reference)
import jax
import jax.numpy as jnp
from jax import lax
from jax.experimental import pallas as pl
from jax.experimental.pallas import tpu as pltpu

N = 10000
N_IN = 128
N_H = 128
B = 400
C = N // B
NBUF = 3
NOBUF = 2


def _gdsa_body(seq_ref, fcwt_ref, adj_hbm, bias_ref, a_ref, linw_ref,
               linb_ref, h_hbm, sc_ref, fts_ref, adj_buf, h_buf,
               in_sems, out_sems):
    fts_ref[...] = jnp.dot(seq_ref[...], fcwt_ref[...],
                           preferred_element_type=jnp.float32)
    a = a_ref[0, 0]
    bias = bias_ref[...]
    linw_r = linw_ref[...].astype(jnp.bfloat16).astype(jnp.float32)
    wsum = jnp.sum(linw_r, axis=0, keepdims=True)
    bsum = jnp.sum(linb_ref[...])

    def start_in(c, slot):
        pltpu.make_async_copy(
            adj_hbm.at[pl.ds(c * B, B), :],
            adj_buf.at[pl.ds(slot * B, B), :],
            in_sems.at[slot],
        ).start()

    for b in range(NBUF):
        start_in(b, b)

    def loop_body(c, carry):
        slot = lax.rem(c, NBUF)
        oslot = lax.rem(c, NOBUF)
        pltpu.make_async_copy(
            adj_hbm.at[pl.ds(c * B, B), :],
            adj_buf.at[pl.ds(slot * B, B), :],
            in_sems.at[slot],
        ).wait()

        out = jnp.dot(adj_buf[pl.ds(slot * B, B), :], fts_ref[...],
                      precision=jax.lax.Precision.DEFAULT,
                      preferred_element_type=jnp.float32)
        out = out + bias
        h = jnp.where(out > 0, out, a * out)

        @pl.when(c >= NOBUF)
        def _():
            pltpu.make_async_copy(
                h_buf.at[pl.ds(oslot * B, B), :],
                h_hbm.at[pl.ds((c - NOBUF) * B, B), :],
                out_sems.at[oslot],
            ).wait()

        h_buf[pl.ds(oslot * B, B), :] = h
        pltpu.make_async_copy(
            h_buf.at[pl.ds(oslot * B, B), :],
            h_hbm.at[pl.ds(c * B, B), :],
            out_sems.at[oslot],
        ).start()

        h_r = h.astype(jnp.bfloat16).astype(jnp.float32)
        sc_ref[pl.ds(c * B, B), :] = (
            jnp.sum(h_r * wsum, axis=1, keepdims=True) + bsum)

        @pl.when(c + NBUF < C)
        def _():
            start_in(c + NBUF, slot)
        return carry

    lax.fori_loop(0, C, loop_body, 0)

    for t in range(NOBUF):
        c = C - NOBUF + t
        slot = c % NOBUF
        pltpu.make_async_copy(
            h_buf.at[pl.ds(slot * B, B), :],
            h_hbm.at[pl.ds(c * B, B), :],
            out_sems.at[slot],
        ).wait()


def kernel(seq, adj, sparse, fc_w, gcn_bias, prelu_a, lin_w, lin_b):
    del sparse
    seq2d = seq.reshape(N, N_IN)
    adj2d = adj.reshape(N, N)
    fcwt = fc_w.T
    bias2d = gcn_bias.reshape(1, N_H)
    a2d = jnp.asarray(prelu_a, jnp.float32).reshape(1, 1)
    linb2d = lin_b.reshape(1, N_H)

    vmem = pltpu.MemorySpace.VMEM
    any_ = pltpu.MemorySpace.HBM
    h2d, sc2d = pl.pallas_call(
        _gdsa_body,
        in_specs=[
            pl.BlockSpec(memory_space=vmem),
            pl.BlockSpec(memory_space=vmem),
            pl.BlockSpec(memory_space=any_),
            pl.BlockSpec(memory_space=vmem),
            pl.BlockSpec(memory_space=vmem),
            pl.BlockSpec(memory_space=vmem),
            pl.BlockSpec(memory_space=vmem),
        ],
        out_specs=[
            pl.BlockSpec(memory_space=any_),
            pl.BlockSpec(memory_space=vmem),
        ],
        out_shape=[
            jax.ShapeDtypeStruct((N, N_H), jnp.float32),
            jax.ShapeDtypeStruct((N, 1), jnp.float32),
        ],
        scratch_shapes=[
            pltpu.VMEM((N, N_H), jnp.float32),
            pltpu.VMEM((NBUF * B, N), jnp.float32),
            pltpu.VMEM((NOBUF * B, N_H), jnp.float32),
            pltpu.SemaphoreType.DMA((NBUF,)),
            pltpu.SemaphoreType.DMA((NOBUF,)),
        ],
        compiler_params=pltpu.CompilerParams(
            vmem_limit_bytes=64 * 1024 * 1024,
        ),
    )(seq2d, fcwt, adj2d, bias2d, a2d, lin_w, linb2d)

    logits = sc2d.reshape(1, N)
    h = h2d.reshape(1, N, N_H)
    return (logits, h)


# speedup vs baseline: 1.0488x; 1.0488x over previous
import jax
import jax.numpy as jnp
from jax.experimental import pallas as pl
from jax.experimental.pallas import tpu as pltpu

N = 10000
N_IN = 128
N_H = 128
BLOCK_M = 400
GRID = (N + BLOCK_M - 1) // BLOCK_M


def _gdsa_body(seq_ref, fcwt_ref, adj_ref, bias_ref, a_ref, linw_ref,
               linb_ref, h_ref, sc_ref, fts_ref):
    i = pl.program_id(0)

    @pl.when(i == 0)
    def _():
        fts = jnp.dot(seq_ref[...], fcwt_ref[...],
                      preferred_element_type=jnp.float32)
        fts_ref[...] = fts

    out = jnp.dot(adj_ref[...], fts_ref[...],
                  precision=jax.lax.Precision.DEFAULT,
                  preferred_element_type=jnp.float32)
    out = out + bias_ref[...]
    a = a_ref[0, 0]
    h = jnp.where(out > 0, out, a * out)
    h_ref[...] = h

    linw_r = linw_ref[...].astype(jnp.bfloat16).astype(jnp.float32)
    wsum = jnp.sum(linw_r, axis=0, keepdims=True)
    bsum = jnp.sum(linb_ref[...])
    h_r = h.astype(jnp.bfloat16).astype(jnp.float32)
    sc_ref[...] = jnp.sum(h_r * wsum, axis=1, keepdims=True) + bsum


def kernel(seq, adj, sparse, fc_w, gcn_bias, prelu_a, lin_w, lin_b):
    del sparse
    seq2d = seq.reshape(N, N_IN)
    adj2d = adj.reshape(N, N)
    fcwt = fc_w.T
    bias2d = gcn_bias.reshape(1, N_H)
    a2d = jnp.asarray(prelu_a, jnp.float32).reshape(1, 1)
    linb2d = lin_b.reshape(1, N_H)

    h2d, sc2d = pl.pallas_call(
        _gdsa_body,
        grid=(GRID,),
        in_specs=[
            pl.BlockSpec((N, N_IN), lambda i: (0, 0)),
            pl.BlockSpec((N_IN, N_H), lambda i: (0, 0)),
            pl.BlockSpec((BLOCK_M, N), lambda i: (i, 0)),
            pl.BlockSpec((1, N_H), lambda i: (0, 0)),
            pl.BlockSpec((1, 1), lambda i: (0, 0)),
            pl.BlockSpec((N_H, N_H), lambda i: (0, 0)),
            pl.BlockSpec((1, N_H), lambda i: (0, 0)),
        ],
        out_specs=[
            pl.BlockSpec((BLOCK_M, N_H), lambda i: (i, 0)),
            pl.BlockSpec((BLOCK_M, 1), lambda i: (i, 0)),
        ],
        out_shape=[
            jax.ShapeDtypeStruct((N, N_H), jnp.float32),
            jax.ShapeDtypeStruct((N, 1), jnp.float32),
        ],
        scratch_shapes=[pltpu.VMEM((N, N_H), jnp.float32)],
        compiler_params=pltpu.CompilerParams(
            dimension_semantics=("arbitrary",),
        ),
    )(seq2d, fcwt, adj2d, bias2d, a2d, lin_w, linb2d)

    logits = sc2d.reshape(1, N)
    h = h2d.reshape(1, N, N_H)
    return (logits, h)
